# baseline (speedup 1.0000x reference)
import functools

import numpy as np
import jax
import jax.numpy as jnp
from jax.experimental import pallas as pl
from jax.experimental.pallas import tpu as pltpu

_SUBLANES = 8


def _dft_mats(H, W):
    n = np.arange(W, dtype=np.float64)[:, None]
    l_re = np.arange(W // 2 + 1, dtype=np.float64)[None, :]
    l_im = np.arange(1, W // 2, dtype=np.float64)[None, :]
    m_mat = np.concatenate(
        [np.cos(2.0 * np.pi * n * l_re / W),
         -np.sin(2.0 * np.pi * n * l_im / W)], axis=1)
    k = np.arange(H, dtype=np.float64)[:, None]
    m = np.arange(H, dtype=np.float64)[None, :]
    ang_h = 2.0 * np.pi * k * m / H
    cs_mat = np.concatenate([np.cos(ang_h), np.sin(ang_h)], axis=0)
    to_bf = lambda x: jnp.asarray(x.astype(np.float32), dtype=jnp.bfloat16)
    return to_bf(m_mat), to_bf(cs_mat)


def _half_batch(B, H, W, p_ref, t_ref, m_ref, cs_ref, w, nz):
    half = W // 2
    x = (p_ref[...] - t_ref[...]).astype(jnp.bfloat16).reshape(B * H, W)
    p = jnp.dot(x, m_ref[...],
                preferred_element_type=jnp.float32).astype(jnp.bfloat16)
    acc = jnp.zeros((H // _SUBLANES, _SUBLANES, half), jnp.float32)
    for b in range(B):
        q = jnp.dot(cs_ref[...], p[b * H:(b + 1) * H, :],
                    preferred_element_type=jnp.float32)
        cp0, cp1 = q[:H, :half], q[:H, half:]
        sp0, sp1 = q[H:, :half], q[H:, half:]
        term = (jnp.abs(cp0 + w * sp1) + jnp.abs(w * cp1 - sp0)
                + jnp.where(nz, jnp.float32(0.0), jnp.abs(cp1) + jnp.abs(sp1)))
        acc = acc + term.reshape(H // _SUBLANES, _SUBLANES, half)
    return acc


def _fused_kernel(B2, H, W, plo_ref, phi_ref, tlo_ref, thi_ref, m_ref, cs_ref,
                  out_ref):
    @pl.when(pl.program_id(1) == 0)
    def _():
        out_ref[...] = jnp.zeros_like(out_ref)

    half = W // 2
    lane = jax.lax.broadcasted_iota(jnp.int32, (H, half), 1)
    nz = lane != 0
    w = jnp.where(nz, jnp.float32(1.0), jnp.float32(0.0))

    acc = (_half_batch(B2, H, W, plo_ref, tlo_ref, m_ref, cs_ref, w, nz)
           + _half_batch(B2, H, W, phi_ref, thi_ref, m_ref, cs_ref, w, nz))
    out_ref[0, :, :] += acc.sum(axis=0)


@jax.jit
def _fft_l1_mean(pred, target):
    N, C, H, W = pred.shape
    n_images = N * C
    n_elems = n_images * H * (W // 2 + 1) * 2

    num_shards = 2 if n_images % 2 == 0 else 1
    per_shard = n_images // num_shards
    batch = 2
    for cand in (24, 16, 8, 4, 2):
        if per_shard % cand == 0:
            batch = cand
            break
    if per_shard % 2 != 0:
        num_shards, per_shard, batch = 1, n_images, 2 if n_images % 2 == 0 else 1
    steps = per_shard // batch
    b2 = batch // 2

    p3 = pred.reshape(n_images, H, W)
    t3 = target.reshape(n_images, H, W)
    m_mat, cs_mat = _dft_mats(H, W)

    lo_spec = pl.BlockSpec((b2, H, W),
                           lambda s, t: (2 * (s * steps + t), 0, 0))
    hi_spec = pl.BlockSpec((b2, H, W),
                           lambda s, t: (2 * (s * steps + t) + 1, 0, 0))
    m_spec = pl.BlockSpec((W, W), lambda s, t: (0, 0))
    cs_spec = pl.BlockSpec((2 * H, H), lambda s, t: (0, 0))
    out_spec = pl.BlockSpec((1, _SUBLANES, W // 2), lambda s, t: (s, 0, 0))

    partials = pl.pallas_call(
        functools.partial(_fused_kernel, b2, H, W),
        out_shape=jax.ShapeDtypeStruct((num_shards, _SUBLANES, W // 2),
                                       jnp.float32),
        grid_spec=pltpu.PrefetchScalarGridSpec(
            num_scalar_prefetch=0,
            grid=(num_shards, steps),
            in_specs=[lo_spec, hi_spec, lo_spec, hi_spec,
                      m_spec, cs_spec],
            out_specs=out_spec),
        compiler_params=pltpu.CompilerParams(
            dimension_semantics=("parallel", "arbitrary")),
    )(p3, p3, t3, t3, m_mat, cs_mat)

    return jnp.sum(partials) / jnp.float32(n_elems)


def kernel(pred, target):
    return _fft_l1_mean(pred, target)


# speedup vs baseline: 1.0264x; 1.0264x over previous
import functools

import numpy as np
import jax
import jax.numpy as jnp
from jax.experimental import pallas as pl
from jax.experimental.pallas import tpu as pltpu

_SUBLANES = 8


def _dft_mats(H, W):
    n = np.arange(W, dtype=np.float64)[:, None]
    l_re = np.arange(W // 2 + 1, dtype=np.float64)[None, :]
    l_im = np.arange(1, W // 2, dtype=np.float64)[None, :]
    m_mat = np.concatenate(
        [np.cos(2.0 * np.pi * n * l_re / W),
         -np.sin(2.0 * np.pi * n * l_im / W)], axis=1)
    k = np.arange(H, dtype=np.float64)[:, None]
    m = np.arange(H, dtype=np.float64)[None, :]
    ang_h = 2.0 * np.pi * k * m / H
    cs_mat = np.concatenate([np.cos(ang_h), np.sin(ang_h)], axis=0)
    to_bf = lambda x: jnp.asarray(x.astype(np.float32), dtype=jnp.bfloat16)
    return to_bf(m_mat), to_bf(cs_mat)


def _fused_kernel(B, H, W, pred_ref, target_ref, m_ref, cs_ref, out_ref):
    @pl.when(pl.program_id(1) == 0)
    def _():
        out_ref[...] = jnp.zeros_like(out_ref)

    x = (pred_ref[...] - target_ref[...]).astype(jnp.bfloat16).reshape(B * H, W)
    p = jnp.dot(x, m_ref[...],
                preferred_element_type=jnp.float32).astype(jnp.bfloat16)

    half = W // 2
    lane = jax.lax.broadcasted_iota(jnp.int32, (H, half), 1)
    nz = lane != 0
    w = jnp.where(nz, jnp.float32(1.0), jnp.float32(0.0))

    acc = jnp.zeros((H // _SUBLANES, _SUBLANES, half), jnp.float32)
    for b in range(B):
        q = jnp.dot(cs_ref[...], p[b * H:(b + 1) * H, :],
                    preferred_element_type=jnp.float32)
        cp0, cp1 = q[:H, :half], q[:H, half:]
        sp0, sp1 = q[H:, :half], q[H:, half:]
        term = (jnp.abs(cp0 + w * sp1) + jnp.abs(w * cp1 - sp0)
                + jnp.where(nz, jnp.float32(0.0), jnp.abs(cp1) + jnp.abs(sp1)))
        acc = acc + term.reshape(H // _SUBLANES, _SUBLANES, half)

    out_ref[0, :, :] += acc.sum(axis=0)


@jax.jit
def _fft_l1_mean(pred, target):
    N, C, H, W = pred.shape
    n_images = N * C
    n_elems = n_images * H * (W // 2 + 1) * 2

    num_shards = 2 if n_images % 2 == 0 else 1
    per_shard = n_images // num_shards
    batch = 1
    for cand in (16, 8, 4, 2):
        if per_shard % cand == 0:
            batch = cand
            break
    steps = per_shard // batch

    p3 = pred.reshape(n_images, H, W)
    t3 = target.reshape(n_images, H, W)
    m_mat, cs_mat = _dft_mats(H, W)

    img_spec = pl.BlockSpec((batch, H, W), lambda s, t: (s * steps + t, 0, 0))
    m_spec = pl.BlockSpec((W, W), lambda s, t: (0, 0))
    cs_spec = pl.BlockSpec((2 * H, H), lambda s, t: (0, 0))
    out_spec = pl.BlockSpec((1, _SUBLANES, W // 2), lambda s, t: (s, 0, 0))

    partials = pl.pallas_call(
        functools.partial(_fused_kernel, batch, H, W),
        out_shape=jax.ShapeDtypeStruct((num_shards, _SUBLANES, W // 2),
                                       jnp.float32),
        grid_spec=pltpu.PrefetchScalarGridSpec(
            num_scalar_prefetch=0,
            grid=(num_shards, steps),
            in_specs=[img_spec, img_spec, m_spec, cs_spec],
            out_specs=out_spec),
        compiler_params=pltpu.CompilerParams(
            dimension_semantics=("parallel", "arbitrary")),
    )(p3, t3, m_mat, cs_mat)

    return jnp.sum(partials) / jnp.float32(n_elems)


def kernel(pred, target):
    return _fft_l1_mean(pred, target)
